# baseline (speedup 1.0000x reference)
import dataclasses

import jax
import jax.numpy as jnp
from jax import lax
from jax.experimental import pallas as pl
from jax.experimental.pallas import tpu as pltpu
from jax.experimental.pallas import tpu_sc as plsc

_N = 10000
_E = 320000
_D = 128
_B = 4
_OUT = 128
_NMAT = 3

_NSUB = 16
_K = 80
_EPT = 20480
_EP = _EPT * _NSUB
_NBLK = _EPT // _K
_NP = 10240
_STRIPE = _NP // _NSUB
_G = 16


def _scale_rows(gb, sb, val_v, g):
    vg = jnp.full((16,), g, jnp.int32)

    @pl.loop(0, _K, step=4)
    def _(e):
        for u in range(4):
            ee = e + u
            sp = plsc.load_gather(
                val_v, [vg, jnp.full((16,), ee, jnp.int32)])
            for q in range(_D // 16):
                sl = pl.ds(q * 16, 16)
                sb[ee, sl] = gb[ee, sl] * sp


def _spmm_phase(xb, zb, src_hbm, dst_hbm, val_hbm,
                src_v, dst_v, val_v, gb_a, gb_b, sb_a, sb_b, acc,
                gs_a, gs_b, ss_a, ss_b, sid):
    @pl.loop(0, _K)
    def _(r):
        for q in range(_D // 16):
            sb_a[r, pl.ds(q * 16, 16)] = jnp.zeros((16,), jnp.float32)

    for z in range(_STRIPE // _K):
        pltpu.sync_copy(sb_a, acc.at[pl.ds(sid * _STRIPE + z * _K, _K)])
    plsc.subcore_barrier()

    @pl.loop(0, _NBLK, step=_G)
    def _(c0):
        pltpu.sync_copy(src_hbm.at[pl.ds(sid * _NBLK + c0, _G)], src_v)
        pltpu.sync_copy(dst_hbm.at[pl.ds(sid * _NBLK + c0, _G)], dst_v)
        pltpu.sync_copy(val_hbm.at[pl.ds(sid * _NBLK + c0, _G)], val_v)

        pltpu.async_copy(xb.at[src_v.at[0]], gb_a, gs_a)
        pltpu.async_copy(xb.at[src_v.at[1]], gb_b, gs_b)

        @pl.loop(0, _G, step=2)
        def _(g):
            pltpu.make_async_copy(xb.at[src_v.at[g]], gb_a, gs_a).wait()

            @pl.when(g >= 2)
            def _():
                pltpu.make_async_copy(sb_a, acc.at[dst_v.at[0]], ss_a).wait()

            _scale_rows(gb_a, sb_a, val_v, g)

            @pl.when(g + 2 < _G)
            def _():
                pltpu.async_copy(xb.at[src_v.at[g + 2]], gb_a, gs_a)

            pltpu.async_copy(sb_a, acc.at[dst_v.at[g]], ss_a, add=True)

            pltpu.make_async_copy(xb.at[src_v.at[g]], gb_b, gs_b).wait()

            @pl.when(g >= 2)
            def _():
                pltpu.make_async_copy(sb_b, acc.at[dst_v.at[0]], ss_b).wait()

            _scale_rows(gb_b, sb_b, val_v, g + 1)

            @pl.when(g + 3 < _G)
            def _():
                pltpu.async_copy(xb.at[src_v.at[g + 3]], gb_b, gs_b)

            pltpu.async_copy(sb_b, acc.at[dst_v.at[g + 1]], ss_b, add=True)

        pltpu.make_async_copy(sb_a, acc.at[dst_v.at[0]], ss_a).wait()
        pltpu.make_async_copy(sb_b, acc.at[dst_v.at[0]], ss_b).wait()

    plsc.subcore_barrier()
    pltpu.sync_copy(acc.at[pl.ds(sid * _STRIPE, _STRIPE)],
                    zb.at[pl.ds(sid * _STRIPE, _STRIPE)])


def _spmm_body(x0, x1, x2, x3, src_hbm, dst_hbm, val_hbm,
               z10, z11, z12, z13, z20, z21, z22, z23,
               src_v, dst_v, val_v, gb_a, gb_b, sb_a, sb_b, acc,
               gs_a, gs_b, ss_a, ss_b):
    cid = lax.axis_index("c")
    sid = lax.axis_index("s")

    xs = (x0, x1, x2, x3)
    z1s = (z10, z11, z12, z13)
    z2s = (z20, z21, z22, z23)
    for p in range(2):
        for b in range(_B):
            @pl.when(cid == b // 2)
            def _():
                xb = xs[b] if p == 0 else z1s[b]
                zb = z1s[b] if p == 0 else z2s[b]
                _spmm_phase(xb, zb, src_hbm, dst_hbm, val_hbm,
                            src_v, dst_v, val_v, gb_a, gb_b, sb_a, sb_b,
                            acc, gs_a, gs_b, ss_a, ss_b, sid)


def _spmm(xs4, src2, dst2, val2):
    mesh = plsc.VectorSubcoreMesh(core_axis_name="c", subcore_axis_name="s")
    out_t = [jax.ShapeDtypeStruct((_NP, _D), jnp.float32) for _ in range(8)]
    scratch = [
        pltpu.VMEM((_G, _K), jnp.int32),
        pltpu.VMEM((_G, _K), jnp.int32),
        pltpu.VMEM((_G, _K), jnp.float32),
        pltpu.VMEM((_K, _D), jnp.float32),
        pltpu.VMEM((_K, _D), jnp.float32),
        pltpu.VMEM((_K, _D), jnp.float32),
        pltpu.VMEM((_K, _D), jnp.float32),
        pltpu.VMEM_SHARED((_NP, _D), jnp.float32),
        pltpu.SemaphoreType.DMA,
        pltpu.SemaphoreType.DMA,
        pltpu.SemaphoreType.DMA,
        pltpu.SemaphoreType.DMA,
    ]
    cp = pltpu.CompilerParams()
    if "needs_layout_passes" in pltpu.CompilerParams.__dataclass_fields__:
        cp = dataclasses.replace(cp, needs_layout_passes=False)
    f = pl.kernel(_spmm_body, out_type=out_t, mesh=mesh,
                  scratch_types=scratch, compiler_params=cp)
    outs = f(*xs4, src2, dst2, val2)
    return outs[:4], outs[4:]


_RB = 1000


def _proj_body(x_ref, z10, z11, z12, z13, z20, z21, z22, z23,
               wa_ref, wb_ref, wc_ref, b_ref, o_ref):
    wa = wa_ref[...]
    wb = wb_ref[...]
    wc = wc_ref[...]
    bias = b_ref[...]
    z1r = (z10, z11, z12, z13)
    z2r = (z20, z21, z22, z23)
    for b in range(_B):
        o_ref[b] = (
            jnp.dot(x_ref[b], wa, preferred_element_type=jnp.float32)
            + jnp.dot(z1r[b][...], wb, preferred_element_type=jnp.float32)
            + jnp.dot(z2r[b][...], wc, preferred_element_type=jnp.float32)
            + bias)


def _proj(inputs, z1s, z2s, wa, wb, wc, bias2):
    grid = (_N // _RB,)
    zspec = pl.BlockSpec((_RB, _D), lambda i: (i, 0))
    wspec = pl.BlockSpec((_D, _OUT), lambda i: (0, 0))
    in_specs = ([pl.BlockSpec((_B, _RB, _D), lambda i: (0, i, 0))]
                + [zspec] * 8 + [wspec] * 3
                + [pl.BlockSpec((1, _OUT), lambda i: (0, 0))])
    out = pl.pallas_call(
        _proj_body,
        grid=grid,
        in_specs=in_specs,
        out_specs=pl.BlockSpec((_B, _RB, _OUT), lambda i: (0, i, 0)),
        out_shape=jax.ShapeDtypeStruct((_B, _N, _OUT), jnp.float32),
    )(inputs, *z1s, *z2s, wa, wb, wc, bias2)
    return out.reshape(_B * _N, _OUT)


def kernel(inputs, edge_index, edge_values, weight, biases):
    dst = edge_index[0].astype(jnp.int32)
    src = edge_index[1].astype(jnp.int32)
    pad = _EP - _E
    pad_idx = jnp.arange(pad, dtype=jnp.int32)
    src2 = jnp.concatenate([src, pad_idx % _N]).reshape(_NSUB * _NBLK, _K)
    dst2 = jnp.concatenate([dst, _N + (pad_idx % (_NP - _N))]
                           ).reshape(_NSUB * _NBLK, _K)
    val2 = jnp.concatenate([edge_values, jnp.zeros((pad,), jnp.float32)]
                           ).reshape(_NSUB * _NBLK, _K)

    xs = tuple(inputs[b] for b in range(_B))
    z1, z2 = _spmm(xs, src2, dst2, val2)

    w = weight.reshape(_D, _NMAT, _OUT)
    wa = w[:, 0, :] - w[:, 2, :]
    wb = w[:, 1, :]
    wc = 2.0 * w[:, 2, :]
    bias2 = biases.reshape(1, _OUT)
    return _proj(inputs, tuple(z1), tuple(z2), wa, wb, wc, bias2)


# speedup vs baseline: 2.5170x; 2.5170x over previous
import dataclasses

import jax
import jax.numpy as jnp
from jax import lax
from jax.experimental import pallas as pl
from jax.experimental.pallas import tpu as pltpu
from jax.experimental.pallas import tpu_sc as plsc

_N = 10000
_E = 320000
_D = 128
_B = 4
_OUT = 128
_NMAT = 3

_NSUB = 16
_K = 128
_EPT = 20480
_EP = _EPT * _NSUB
_NBLK = _EPT // _K
_NP = 10240
_STRIPE = _NP // _NSUB
_G = 40


def _scale_rows(rows, val_v, g):
    vg = jnp.full((16,), g, jnp.int32)

    @pl.loop(0, _K, step=4)
    def _(e):
        for u in range(4):
            ee = e + u
            sp = plsc.load_gather(
                val_v, [vg, jnp.full((16,), ee, jnp.int32)])
            for q in range(_D // 16):
                sl = pl.ds(q * 16, 16)
                rows[ee, sl] = rows[ee, sl] * sp


def _spmm_phase(xb, zb, src_hbm, dst_hbm, val_hbm,
                src_v, dst_v, val_v, rows_a, rows_b, acc,
                gs_a, gs_b, ss_a, ss_b, sid):
    @pl.loop(0, _K)
    def _(r):
        for q in range(_D // 16):
            rows_a[r, pl.ds(q * 16, 16)] = jnp.zeros((16,), jnp.float32)

    for z in range(_STRIPE // _K):
        pltpu.sync_copy(rows_a, acc.at[pl.ds(sid * _STRIPE + z * _K, _K)])
    plsc.subcore_barrier()

    @pl.loop(0, _NBLK, step=_G)
    def _(c0):
        pltpu.sync_copy(src_hbm.at[pl.ds(sid * _NBLK + c0, _G)], src_v)
        pltpu.sync_copy(dst_hbm.at[pl.ds(sid * _NBLK + c0, _G)], dst_v)
        pltpu.sync_copy(val_hbm.at[pl.ds(sid * _NBLK + c0, _G)], val_v)

        pltpu.async_copy(xb.at[src_v.at[0]], rows_a, gs_a)
        pltpu.async_copy(xb.at[src_v.at[1]], rows_b, gs_b)

        @pl.loop(0, _G, step=2)
        def _(g):
            pltpu.make_async_copy(xb.at[src_v.at[g]], rows_a, gs_a).wait()
            _scale_rows(rows_a, val_v, g)
            pltpu.async_copy(rows_a, acc.at[dst_v.at[g]], ss_a, add=True)

            @pl.when(g + 2 < _G)
            def _():
                pltpu.make_async_copy(
                    rows_a, acc.at[dst_v.at[g]], ss_a).wait()
                pltpu.async_copy(xb.at[src_v.at[g + 2]], rows_a, gs_a)

            pltpu.make_async_copy(xb.at[src_v.at[g]], rows_b, gs_b).wait()
            _scale_rows(rows_b, val_v, g + 1)
            pltpu.async_copy(rows_b, acc.at[dst_v.at[g + 1]], ss_b, add=True)

            @pl.when(g + 3 < _G)
            def _():
                pltpu.make_async_copy(
                    rows_b, acc.at[dst_v.at[g]], ss_b).wait()
                pltpu.async_copy(xb.at[src_v.at[g + 3]], rows_b, gs_b)

        pltpu.make_async_copy(rows_a, acc.at[dst_v.at[0]], ss_a).wait()
        pltpu.make_async_copy(rows_b, acc.at[dst_v.at[0]], ss_b).wait()

    plsc.subcore_barrier()
    pltpu.sync_copy(acc.at[pl.ds(sid * _STRIPE, _STRIPE)],
                    zb.at[pl.ds(sid * _STRIPE, _STRIPE)])


def _spmm_body(x0, x1, x2, x3, src_hbm, dst_hbm, val_hbm,
               z10, z11, z12, z13, z20, z21, z22, z23,
               src_v, dst_v, val_v, rows_a, rows_b, acc,
               gs_a, gs_b, ss_a, ss_b):
    cid = lax.axis_index("c")
    sid = lax.axis_index("s")

    xs = (x0, x1, x2, x3)
    z1s = (z10, z11, z12, z13)
    z2s = (z20, z21, z22, z23)
    for p in range(2):
        for b in range(_B):
            @pl.when(cid == b // 2)
            def _():
                xb = xs[b] if p == 0 else z1s[b]
                zb = z1s[b] if p == 0 else z2s[b]
                _spmm_phase(xb, zb, src_hbm, dst_hbm, val_hbm,
                            src_v, dst_v, val_v, rows_a, rows_b,
                            acc, gs_a, gs_b, ss_a, ss_b, sid)


def _spmm(xs4, src2, dst2, val2):
    mesh = plsc.VectorSubcoreMesh(core_axis_name="c", subcore_axis_name="s")
    out_t = [jax.ShapeDtypeStruct((_NP, _D), jnp.float32) for _ in range(8)]
    scratch = [
        pltpu.VMEM((_G, _K), jnp.int32),
        pltpu.VMEM((_G, _K), jnp.int32),
        pltpu.VMEM((_G, _K), jnp.float32),
        pltpu.VMEM((_K, _D), jnp.float32),
        pltpu.VMEM((_K, _D), jnp.float32),
        pltpu.VMEM_SHARED((_NP, _D), jnp.float32),
        pltpu.SemaphoreType.DMA,
        pltpu.SemaphoreType.DMA,
        pltpu.SemaphoreType.DMA,
        pltpu.SemaphoreType.DMA,
    ]
    cp = pltpu.CompilerParams()
    if "needs_layout_passes" in pltpu.CompilerParams.__dataclass_fields__:
        cp = dataclasses.replace(cp, needs_layout_passes=False)
    f = pl.kernel(_spmm_body, out_type=out_t, mesh=mesh,
                  scratch_types=scratch, compiler_params=cp)
    outs = f(*xs4, src2, dst2, val2)
    return outs[:4], outs[4:]


_RB = 1000


def _proj_body(x_ref, z10, z11, z12, z13, z20, z21, z22, z23,
               wa_ref, wb_ref, wc_ref, b_ref, o_ref):
    wa = wa_ref[...]
    wb = wb_ref[...]
    wc = wc_ref[...]
    bias = b_ref[...]
    z1r = (z10, z11, z12, z13)
    z2r = (z20, z21, z22, z23)
    for b in range(_B):
        o_ref[b] = (
            jnp.dot(x_ref[b], wa, preferred_element_type=jnp.float32)
            + jnp.dot(z1r[b][...], wb, preferred_element_type=jnp.float32)
            + jnp.dot(z2r[b][...], wc, preferred_element_type=jnp.float32)
            + bias)


def _proj(inputs, z1s, z2s, wa, wb, wc, bias2):
    grid = (_N // _RB,)
    zspec = pl.BlockSpec((_RB, _D), lambda i: (i, 0))
    wspec = pl.BlockSpec((_D, _OUT), lambda i: (0, 0))
    in_specs = ([pl.BlockSpec((_B, _RB, _D), lambda i: (0, i, 0))]
                + [zspec] * 8 + [wspec] * 3
                + [pl.BlockSpec((1, _OUT), lambda i: (0, 0))])
    out = pl.pallas_call(
        _proj_body,
        grid=grid,
        in_specs=in_specs,
        out_specs=pl.BlockSpec((_B, _RB, _OUT), lambda i: (0, i, 0)),
        out_shape=jax.ShapeDtypeStruct((_B, _N, _OUT), jnp.float32),
    )(inputs, *z1s, *z2s, wa, wb, wc, bias2)
    return out.reshape(_B * _N, _OUT)


def kernel(inputs, edge_index, edge_values, weight, biases):
    dst = edge_index[0].astype(jnp.int32)
    src = edge_index[1].astype(jnp.int32)
    pad = _EP - _E
    pad_idx = jnp.arange(pad, dtype=jnp.int32)
    src2 = jnp.concatenate([src, pad_idx % _N]).reshape(_NSUB * _NBLK, _K)
    dst2 = jnp.concatenate([dst, _N + (pad_idx % (_NP - _N))]
                           ).reshape(_NSUB * _NBLK, _K)
    val2 = jnp.concatenate([edge_values, jnp.zeros((pad,), jnp.float32)]
                           ).reshape(_NSUB * _NBLK, _K)

    xs = tuple(inputs[b] for b in range(_B))
    z1, z2 = _spmm(xs, src2, dst2, val2)

    w = weight.reshape(_D, _NMAT, _OUT)
    wa = w[:, 0, :] - w[:, 2, :]
    wb = w[:, 1, :]
    wc = 2.0 * w[:, 2, :]
    bias2 = biases.reshape(1, _OUT)
    return _proj(inputs, tuple(z1), tuple(z2), wa, wb, wc, bias2)
